# baseline (speedup 1.0000x reference)
import functools

import jax
import jax.numpy as jnp
from jax.experimental import pallas as pl
from jax.experimental.pallas import tpu as pltpu

_E = 8
_LANES = 128
_NEG_INF = float("-inf")
_TILE = 512


def _moe_body(x_ref, gwt_ref, bpad_ref, a_hbm, out_ref, a_vmem, xw_ref,
              sems):
    t = pl.program_id(0)

    @pl.when(t == 0)
    def _prefetch_a():
        for e in range(_E):
            pltpu.make_async_copy(a_hbm.at[e], a_vmem.at[e],
                                  sems.at[e]).start()

    xt = x_ref[...]
    logits = jnp.dot(xt, gwt_ref[...],
                     preferred_element_type=jnp.float32)
    lane = jax.lax.broadcasted_iota(jnp.int32, logits.shape, 1)
    logits = jnp.where(lane < _E, logits, _NEG_INF)
    m1 = jnp.max(logits, axis=1, keepdims=True)
    idx0 = jnp.min(jnp.where(logits == m1, lane, _LANES), axis=1,
                   keepdims=True)
    logits2 = jnp.where(lane == idx0, _NEG_INF, logits)
    m2 = jnp.max(logits2, axis=1, keepdims=True)
    idx1 = jnp.min(jnp.where(logits2 == m2, lane, _LANES), axis=1,
                   keepdims=True)
    s = jnp.exp(m2 - m1)
    w0 = 1.0 / (1.0 + s)
    w1 = 1.0 - w0
    dense_w = (jnp.where(lane == idx0, w0, 0.0)
               + jnp.where(lane == idx1, w1, 0.0))
    for e in range(_E):
        xw_ref[e] = (dense_w[:, e:e + 1] * xt).astype(jnp.bfloat16)

    acc = jnp.dot(dense_w, bpad_ref[...],
                  preferred_element_type=jnp.float32)
    for e in range(_E):
        @pl.when(t == 0)
        def _wait(e=e):
            pltpu.make_async_copy(a_hbm.at[e], a_vmem.at[e],
                                  sems.at[e]).wait()
        acc += jnp.dot(xw_ref[e], a_vmem[e],
                       preferred_element_type=jnp.float32)
    out_ref[...] = acc


@functools.partial(jax.jit, static_argnames=())
def kernel(inputs, gate_w, expert_A, expert_b):
    batch_shape = inputs.shape[:-1]
    d = inputs.shape[-1]
    x = inputs.reshape(-1, d)
    t = x.shape[0]

    gwt = jnp.zeros((d, _LANES), dtype=gate_w.dtype).at[:, :_E].set(gate_w.T)
    bpad = jnp.zeros((_LANES, d), dtype=expert_b.dtype).at[:_E].set(expert_b)

    out = pl.pallas_call(
        _moe_body,
        grid=(t // _TILE,),
        in_specs=[
            pl.BlockSpec((_TILE, d), lambda i: (i, 0)),
            pl.BlockSpec((d, _LANES), lambda i: (0, 0)),
            pl.BlockSpec((_LANES, d), lambda i: (0, 0)),
            pl.BlockSpec(memory_space=pl.ANY),
        ],
        out_specs=pl.BlockSpec((_TILE, d), lambda i: (i, 0)),
        out_shape=jax.ShapeDtypeStruct((t, d), jnp.float32),
        scratch_shapes=[
            pltpu.VMEM((_E, d, d), jnp.float32),
            pltpu.VMEM((_E, _TILE, d), jnp.bfloat16),
            pltpu.SemaphoreType.DMA((_E,)),
        ],
    )(x, gwt, bpad, expert_A)
    return out.reshape(*batch_shape, d)


# speedup vs baseline: 1.2297x; 1.2297x over previous
import functools

import jax
import jax.numpy as jnp
from jax.experimental import pallas as pl
from jax.experimental.pallas import tpu as pltpu

_E = 8
_LANES = 128
_NEG_INF = float("-inf")
_TILE = 1024


def _moe_body(x_ref, gwt_ref, bpad_ref, a_ref, out_ref, abf_ref, xw_ref):
    t = pl.program_id(0)

    @pl.when(t == 0)
    def _cast_a():
        for e in range(_E):
            abf_ref[e] = a_ref[e].astype(jnp.bfloat16)

    xt = x_ref[...]
    xb = xt.astype(jnp.bfloat16)
    logits = jnp.dot(xt, gwt_ref[...],
                     preferred_element_type=jnp.float32)
    lane = jax.lax.broadcasted_iota(jnp.int32, logits.shape, 1)
    logits = jnp.where(lane < _E, logits, _NEG_INF)
    m1 = jnp.max(logits, axis=1, keepdims=True)
    idx0 = jnp.min(jnp.where(logits == m1, lane, _LANES), axis=1,
                   keepdims=True)
    logits2 = jnp.where(lane == idx0, _NEG_INF, logits)
    m2 = jnp.max(logits2, axis=1, keepdims=True)
    idx1 = jnp.min(jnp.where(logits2 == m2, lane, _LANES), axis=1,
                   keepdims=True)
    s = jnp.exp(m2 - m1)
    w0 = 1.0 / (1.0 + s)
    w1 = 1.0 - w0
    dense_w = (jnp.where(lane == idx0, w0, 0.0)
               + jnp.where(lane == idx1, w1, 0.0))
    dense_wb = dense_w.astype(jnp.bfloat16)
    for e in range(_E):
        xw_ref[e] = dense_wb[:, e:e + 1] * xb

    acc = jnp.dot(dense_w, bpad_ref[...],
                  preferred_element_type=jnp.float32)
    for e in range(_E):
        acc += jnp.dot(xw_ref[e], abf_ref[e],
                       preferred_element_type=jnp.float32)
    out_ref[...] = acc


@functools.partial(jax.jit, static_argnames=())
def kernel(inputs, gate_w, expert_A, expert_b):
    batch_shape = inputs.shape[:-1]
    d = inputs.shape[-1]
    x = inputs.reshape(-1, d)
    t = x.shape[0]

    gwt = jnp.zeros((d, _LANES), dtype=gate_w.dtype).at[:, :_E].set(gate_w.T)
    bpad = jnp.zeros((_LANES, d), dtype=expert_b.dtype).at[:_E].set(expert_b)

    out = pl.pallas_call(
        _moe_body,
        grid=(t // _TILE,),
        in_specs=[
            pl.BlockSpec((_TILE, d), lambda i: (i, 0)),
            pl.BlockSpec((d, _LANES), lambda i: (0, 0)),
            pl.BlockSpec((_LANES, d), lambda i: (0, 0)),
            pl.BlockSpec((_E, d, d), lambda i: (0, 0, 0)),
        ],
        out_specs=pl.BlockSpec((_TILE, d), lambda i: (i, 0)),
        out_shape=jax.ShapeDtypeStruct((t, d), jnp.float32),
        scratch_shapes=[
            pltpu.VMEM((_E, d, d), jnp.bfloat16),
            pltpu.VMEM((_E, _TILE, d), jnp.bfloat16),
        ],
    )(x, gwt, bpad, expert_A)
    return out.reshape(*batch_shape, d)
